# baseline (speedup 1.0000x reference)
import functools

import jax
import jax.numpy as jnp
from jax.experimental import pallas as pl
from jax.experimental.pallas import tpu as pltpu

B, MAX_ATOM, N_FEAT = 64, 128, 64
IB = 8


def _pool_kernel(nslice_ref, x_ref, l_ref, out_ref):
    b = pl.program_id(0)
    ib = pl.program_id(1)
    m = nslice_ref[b, 0]

    x = x_ref[0]
    l_rows = l_ref[0]

    row_ids = ib * IB + jax.lax.broadcasted_iota(jnp.int32, (IB, MAX_ATOM), 0)
    col_ids = jax.lax.broadcasted_iota(jnp.int32, (IB, MAX_ATOM), 1)
    adj = (l_rows != 0.0) & (row_ids < m) & (col_ids < m)

    bias = jnp.where(adj, 0.0, -1e30)
    masked = x[None, :, :] + bias[:, :, None]
    pooled = jnp.max(masked, axis=1)

    has_nb = jnp.any(adj, axis=1)
    x_rows = x_ref[0, pl.ds(ib * IB, IB), :]
    pooled = jnp.where(has_nb[:, None], pooled, x_rows)
    row_valid = (ib * IB + jax.lax.broadcasted_iota(jnp.int32, (IB, N_FEAT), 0)) < m
    out_ref[0] = jnp.where(row_valid, pooled, 0.0)


@functools.partial(jax.jit, static_argnames=("interpret",))
def kernel(node_features, original_laplacian, data_slice, lap_slice,
           interpret=False):
    del lap_slice
    grid = (B, MAX_ATOM // IB)
    out = pl.pallas_call(
        _pool_kernel,
        grid=grid,
        in_specs=[
            pl.BlockSpec(memory_space=pltpu.SMEM),
            pl.BlockSpec((1, MAX_ATOM, N_FEAT), lambda b, ib: (b, 0, 0)),
            pl.BlockSpec((1, IB, MAX_ATOM), lambda b, ib: (b, ib, 0)),
        ],
        out_specs=pl.BlockSpec((1, IB, N_FEAT), lambda b, ib: (b, ib, 0)),
        out_shape=jax.ShapeDtypeStruct((B, MAX_ATOM, N_FEAT), jnp.float32),
        compiler_params=pltpu.CompilerParams(
            dimension_semantics=("arbitrary", "arbitrary"),
        ),
        interpret=interpret,
    )(data_slice, node_features, original_laplacian)
    return out


# speedup vs baseline: 6.8000x; 6.8000x over previous
import jax
import jax.numpy as jnp
from jax import lax
from jax.experimental import pallas as pl
from jax.experimental.pallas import tpu as pltpu
from jax.experimental.pallas import tpu_sc as plsc

B, MAX_ATOM, N_FEAT = 64, 128, 64
NC, NS, LANES = 2, 16, 16
NW = NC * NS
MOLS_PER_W = B // NW
NCHUNK = MAX_ATOM // LANES
NFG = N_FEAT // LANES

_NEG = -1e30


def _sc_body(x_hbm, l_hbm, n_hbm, out_hbm, l_v, x_v, o_v, nbr_v, m_v):
    cid = lax.axis_index("c")
    sid = lax.axis_index("s")
    wid = sid * NC + cid

    lane = jnp.arange(LANES, dtype=jnp.int32)

    for m in range(MOLS_PER_W):
        b = wid * MOLS_PER_W + m
        pltpu.sync_copy(l_hbm.at[b], l_v)
        pltpu.sync_copy(x_hbm.at[b], x_v)
        pltpu.sync_copy(n_hbm.at[b], m_v)
        M = m_v[...][0]

        def row_body(i, carry, M=M):
            off = jnp.zeros((LANES,), jnp.int32)
            for c in range(NCHUNK):
                v = l_v[i, pl.ds(c * LANES, LANES)]
                col = lane + c * LANES
                msk = (v != 0.0) & (col < M)
                mi = jnp.where(msk, 1, 0).astype(jnp.int32)
                pos = plsc.cumsum(mi) - mi + off
                plsc.store_scatter(nbr_v, [pos], col, mask=msk)
                off = off + plsc.all_reduce_population_count(msk)
            deg = off[0]

            def nb_body(d, accs):
                j = nbr_v[pl.ds(d, LANES)][0]
                return tuple(
                    jnp.maximum(accs[g], x_v[j, pl.ds(g * LANES, LANES)])
                    for g in range(NFG)
                )

            accs = tuple(jnp.full((LANES,), _NEG, jnp.float32)
                         for _ in range(NFG))
            accs = lax.fori_loop(0, deg, nb_body, accs)

            has_nb = deg > 0
            valid = i < M
            for g in range(NFG):
                xg = x_v[i, pl.ds(g * LANES, LANES)]
                og = jnp.where(has_nb, accs[g], xg)
                og = jnp.where(valid, og, 0.0)
                o_v[i, pl.ds(g * LANES, LANES)] = og
            return carry

        lax.fori_loop(0, MAX_ATOM, row_body, 0)
        pltpu.sync_copy(o_v, out_hbm.at[b])


@jax.jit
def kernel(node_features, original_laplacian, data_slice, lap_slice):
    del lap_slice
    natoms = jnp.broadcast_to(data_slice[:, :1], (B, LANES)).astype(jnp.int32)
    mesh = plsc.VectorSubcoreMesh(core_axis_name="c", subcore_axis_name="s")
    run = pl.kernel(
        _sc_body,
        out_type=jax.ShapeDtypeStruct((B, MAX_ATOM, N_FEAT), jnp.float32),
        mesh=mesh,
        compiler_params=pltpu.CompilerParams(needs_layout_passes=False),
        scratch_types=[
            pltpu.VMEM((MAX_ATOM, MAX_ATOM), jnp.float32),
            pltpu.VMEM((MAX_ATOM, N_FEAT), jnp.float32),
            pltpu.VMEM((MAX_ATOM, N_FEAT), jnp.float32),
            pltpu.VMEM((MAX_ATOM + LANES,), jnp.int32),
            pltpu.VMEM((LANES,), jnp.int32),
        ],
    )
    return run(node_features, original_laplacian, natoms)


# speedup vs baseline: 7.8930x; 1.1607x over previous
import jax
import jax.numpy as jnp
from jax import lax
from jax.experimental import pallas as pl
from jax.experimental.pallas import tpu as pltpu
from jax.experimental.pallas import tpu_sc as plsc

B, MAX_ATOM, N_FEAT = 64, 128, 64
NC, NS, LANES = 2, 16, 16
NW = NC * NS
MOLS_PER_W = B // NW
NCHUNK = MAX_ATOM // LANES
NFG = N_FEAT // LANES

_NEG = -1e30


def _sc_body(x_hbm, l_hbm, n_hbm, out_hbm, l_v, x_v, o_v, nbr_v, m_v):
    cid = lax.axis_index("c")
    sid = lax.axis_index("s")
    wid = sid * NC + cid

    lane = jnp.arange(LANES, dtype=jnp.int32)

    for m in range(MOLS_PER_W):
        b = wid * MOLS_PER_W + m
        pltpu.sync_copy(l_hbm.at[b], l_v)
        pltpu.sync_copy(x_hbm.at[b], x_v)
        pltpu.sync_copy(n_hbm.at[b], m_v)
        M = m_v[...][0]

        nchunks = (M + LANES - 1) // LANES

        def row_body(i, carry, M=M, nchunks=nchunks):
            def chunk_body(c, off):
                v = l_v[i, pl.ds(c * LANES, LANES)]
                col = lane + c * LANES
                msk = (v != 0.0) & (col < M)
                mi = jnp.where(msk, 1, 0)
                pos = plsc.cumsum(mi) - mi + off
                plsc.store_scatter(nbr_v, [pos], col, mask=msk)
                return off + plsc.all_reduce_population_count(msk)

            off = lax.fori_loop(0, nchunks, chunk_body,
                                jnp.zeros((LANES,), jnp.int32))
            deg = off[0]

            first = nbr_v[pl.ds(0, LANES)][0]
            nbr_v[pl.ds(deg, LANES)] = jnp.full((LANES,), first, jnp.int32)

            def quad_body(q, accs):
                jv = nbr_v[pl.ds(q * 4, LANES)]
                accs = list(accs)
                for k in range(4):
                    j = jv[k]
                    for g in range(NFG):
                        accs[g] = jnp.maximum(
                            accs[g], x_v[j, pl.ds(g * LANES, LANES)])
                return tuple(accs)

            accs = tuple(jnp.full((LANES,), _NEG, jnp.float32)
                         for _ in range(NFG))
            accs = lax.fori_loop(0, (deg + 3) // 4, quad_body, accs)

            has_nb = deg > 0
            for g in range(NFG):
                xg = x_v[i, pl.ds(g * LANES, LANES)]
                og = jnp.where(has_nb, accs[g], xg)
                o_v[i, pl.ds(g * LANES, LANES)] = og
            return carry

        def zero_body(i, carry):
            zeros = jnp.zeros((LANES,), jnp.float32)
            for g in range(NFG):
                o_v[i, pl.ds(g * LANES, LANES)] = zeros
            return carry

        lax.fori_loop(0, M, row_body, 0)
        lax.fori_loop(M, MAX_ATOM, zero_body, 0)
        pltpu.sync_copy(o_v, out_hbm.at[b])


@jax.jit
def kernel(node_features, original_laplacian, data_slice, lap_slice):
    del lap_slice
    natoms = jnp.broadcast_to(data_slice[:, :1], (B, LANES)).astype(jnp.int32)
    mesh = plsc.VectorSubcoreMesh(core_axis_name="c", subcore_axis_name="s")
    run = pl.kernel(
        _sc_body,
        out_type=jax.ShapeDtypeStruct((B, MAX_ATOM, N_FEAT), jnp.float32),
        mesh=mesh,
        compiler_params=pltpu.CompilerParams(needs_layout_passes=False),
        scratch_types=[
            pltpu.VMEM((MAX_ATOM, MAX_ATOM), jnp.float32),
            pltpu.VMEM((MAX_ATOM, N_FEAT), jnp.float32),
            pltpu.VMEM((MAX_ATOM, N_FEAT), jnp.float32),
            pltpu.VMEM((MAX_ATOM + LANES,), jnp.int32),
            pltpu.VMEM((LANES,), jnp.int32),
        ],
    )
    return run(node_features, original_laplacian, natoms)


# speedup vs baseline: 8.2341x; 1.0432x over previous
import jax
import jax.numpy as jnp
from jax import lax
from jax.experimental import pallas as pl
from jax.experimental.pallas import tpu as pltpu
from jax.experimental.pallas import tpu_sc as plsc

B, MAX_ATOM, N_FEAT = 64, 128, 64
NC, NS, LANES = 2, 16, 16
NW = NC * NS
MOLS_PER_W = B // NW
NCHUNK = MAX_ATOM // LANES
NFG = N_FEAT // LANES

_NEG = -1e30


def _sc_body(x_hbm, l_hbm, n_hbm, out_hbm, l_v, x_v, o_v, nbr_v, m_v):
    cid = lax.axis_index("c")
    sid = lax.axis_index("s")
    wid = sid * NC + cid

    lane = jnp.arange(LANES, dtype=jnp.int32)

    for m in range(MOLS_PER_W):
        b = wid * MOLS_PER_W + m
        pltpu.sync_copy(l_hbm.at[b], l_v)
        pltpu.sync_copy(x_hbm.at[b], x_v)
        pltpu.sync_copy(n_hbm.at[b], m_v)
        M = m_v[...][0]

        nchunks = (M + LANES - 1) // LANES

        def row_body(i, carry, M=M, nchunks=nchunks):
            def chunk_body(c, off):
                v = l_v[i, pl.ds(c * LANES, LANES)]
                col = lane + c * LANES
                msk = (v != 0.0) & (col < M)
                plsc.store_compressed(nbr_v.at[pl.ds(off, LANES)], col,
                                      mask=msk)
                return off + plsc.all_reduce_population_count(msk)[0]

            deg = lax.fori_loop(0, nchunks, chunk_body, 0)

            def quad_body(q, accs):
                jv = nbr_v[pl.ds(q * 4, LANES)]
                accs = list(accs)
                for k in range(4):
                    ok = q * 4 + k < deg
                    j = jnp.where(ok, jv[k], 0)
                    for g in range(NFG):
                        accs[g] = jnp.where(
                            ok,
                            jnp.maximum(accs[g],
                                        x_v[j, pl.ds(g * LANES, LANES)]),
                            accs[g])
                return tuple(accs)

            accs = tuple(jnp.full((LANES,), _NEG, jnp.float32)
                         for _ in range(NFG))
            accs = lax.fori_loop(0, (deg + 3) // 4, quad_body, accs)

            has_nb = deg > 0
            for g in range(NFG):
                xg = x_v[i, pl.ds(g * LANES, LANES)]
                og = jnp.where(has_nb, accs[g], xg)
                o_v[i, pl.ds(g * LANES, LANES)] = og
            return carry

        def zero_body(i, carry):
            zeros = jnp.zeros((LANES,), jnp.float32)
            for g in range(NFG):
                o_v[i, pl.ds(g * LANES, LANES)] = zeros
            return carry

        lax.fori_loop(0, M, row_body, 0)
        lax.fori_loop(M, MAX_ATOM, zero_body, 0)
        pltpu.sync_copy(o_v, out_hbm.at[b])


@jax.jit
def kernel(node_features, original_laplacian, data_slice, lap_slice):
    del lap_slice
    natoms = jnp.broadcast_to(data_slice[:, :1], (B, LANES)).astype(jnp.int32)
    mesh = plsc.VectorSubcoreMesh(core_axis_name="c", subcore_axis_name="s")
    run = pl.kernel(
        _sc_body,
        out_type=jax.ShapeDtypeStruct((B, MAX_ATOM, N_FEAT), jnp.float32),
        mesh=mesh,
        compiler_params=pltpu.CompilerParams(needs_layout_passes=False),
        scratch_types=[
            pltpu.VMEM((MAX_ATOM, MAX_ATOM), jnp.float32),
            pltpu.VMEM((MAX_ATOM, N_FEAT), jnp.float32),
            pltpu.VMEM((MAX_ATOM, N_FEAT), jnp.float32),
            pltpu.VMEM((MAX_ATOM + LANES,), jnp.int32),
            pltpu.VMEM((LANES,), jnp.int32),
        ],
    )
    return run(node_features, original_laplacian, natoms)
